# baseline (speedup 1.0000x reference)
import functools

import jax
import jax.numpy as jnp
from jax import lax
from jax.experimental import pallas as pl
from jax.experimental.pallas import tpu as pltpu
from jax.experimental.pallas import tpu_sc as plsc

K = 16
BQ = 256
CPAD = 8
SCR = 2048
NW = 32


def _knn_block(pts_nc_ref, pts_cn_ref, idx_ref, *, n):
    b = pl.program_id(0) // (n // BQ)
    q = pts_nc_ref[0]
    p = pts_cn_ref[0]
    sq_q = jnp.sum(q * q, axis=1, keepdims=True)
    sq_p = jnp.sum(p * p, axis=0, keepdims=True)
    inner = jax.lax.dot_general(
        q, p, (((1,), (0,)), ((), ())),
        preferred_element_type=jnp.float32)
    d = sq_q + sq_p - 2.0 * inner
    iota = jax.lax.broadcasted_iota(jnp.int32, (BQ, n), 1).astype(jnp.float32)
    nf = jnp.float32(n)
    cols = []
    for _ in range(K):
        m = jnp.min(d, axis=1, keepdims=True)
        cand = jnp.where(d <= m, iota, nf)
        a = jnp.min(cand, axis=1, keepdims=True)
        cols.append(a)
        d = jnp.where(iota == a, jnp.float32(jnp.inf), d)
    idx = jnp.concatenate(cols, axis=1).astype(jnp.int32)
    idx_ref[0] = idx + b * n


def _tc_knn(points, nblk):
    B, N, C = points.shape
    bpb = N // BQ
    pts_nc = jnp.pad(points, ((0, 0), (0, 0), (0, CPAD - C)))
    pts_cn = jnp.transpose(pts_nc, (0, 2, 1))
    return pl.pallas_call(
        functools.partial(_knn_block, n=N),
        grid=(nblk,),
        in_specs=[
            pl.BlockSpec((1, BQ, CPAD), lambda t: (t // bpb, t % bpb, 0)),
            pl.BlockSpec((1, CPAD, N), lambda t: (t // bpb, 0, 0)),
        ],
        out_specs=pl.BlockSpec((1, BQ, K), lambda t: (t // bpb, t % bpb, 0)),
        out_shape=jax.ShapeDtypeStruct((B, N, K), jnp.int32),
    )(pts_nc, pts_cn)


def _sc_knn(xb, yb, zb, sq, b0, scr, n):
    rpw = scr // NW
    nchunk = n // 16
    mesh = plsc.VectorSubcoreMesh(core_axis_name="c", subcore_axis_name="s")

    @functools.partial(
        pl.kernel, mesh=mesh,
        out_type=jax.ShapeDtypeStruct((scr * K,), jnp.int32),
        scratch_types=[
            pltpu.VMEM((n,), jnp.float32),
            pltpu.VMEM((n,), jnp.float32),
            pltpu.VMEM((n,), jnp.float32),
            pltpu.VMEM((n,), jnp.float32),
            pltpu.VMEM((rpw * K,), jnp.int32),
            pltpu.VMEM((K,), jnp.float32),
            pltpu.VMEM((K,), jnp.int32),
        ],
    )
    def sc_kernel(xb_hbm, yb_hbm, zb_hbm, sq_hbm, out_hbm,
                  xv, yv, zv, sv, ov, bdv, biv):
        wid = lax.axis_index("s") * 2 + lax.axis_index("c")
        pltpu.sync_copy(xb_hbm, xv)
        pltpu.sync_copy(yb_hbm, yv)
        pltpu.sync_copy(zb_hbm, zv)
        pltpu.sync_copy(sq_hbm, sv)
        r0 = (n - scr) + wid * rpw
        goff = b0 * n

        def row_body(rl, carry):
            r = r0 + rl
            g16 = pl.multiple_of((r // 16) * 16, 16)
            lane16 = lax.iota(jnp.int32, 16)
            lane = jnp.broadcast_to(r - g16, (16,))
            qx = xv[pl.ds(g16, 16)][lane]
            qy = yv[pl.ds(g16, 16)][lane]
            qz = zv[pl.ds(g16, 16)][lane]
            sqq = sv[pl.ds(g16, 16)][lane]

            ones = jnp.broadcast_to(1, (16,))
            zeros = jnp.broadcast_to(0, (16,))

            def b2i(m):
                return jnp.where(m, ones, zeros)

            def lex_lt_i(kk, kb, vv, vb):
                return b2i(kk < kb) | (b2i(kk == kb) & b2i(vv < vb))

            def sort16(kk, vv):
                for kbit in (2, 4, 8, 16):
                    s = kbit // 2
                    while s >= 1:
                        prt = jnp.bitwise_xor(lane16, s)
                        kb = kk[prt]
                        vb = vv[prt]
                        lt = lex_lt_i(kk, kb, vv, vb)
                        mm = ones ^ b2i((lane16 & s) == 0) ^ b2i((lane16 & kbit) == 0)
                        keep = (mm ^ lt) == 0
                        kk = jnp.where(keep, kk, kb)
                        vv = jnp.where(keep, vv, vb)
                        s //= 2
                return kk, vv

            def chunk_body(j, carry):
                o = pl.multiple_of(j * 16, 16)
                px = xv[pl.ds(o, 16)]
                py = yv[pl.ds(o, 16)]
                pz = zv[pl.ds(o, 16)]
                sp = sv[pl.ds(o, 16)]
                inner = (qx * px + qy * py) + qz * pz
                d = (sqq + sp) - 2.0 * inner
                mn = d
                for s in (8, 4, 2, 1):
                    mn = jnp.minimum(mn, mn[jnp.bitwise_xor(lane16, s)])

                @pl.when(mn[0] < bdv[...][15])
                def _merge():
                    ck, cv = sort16(d, j * 16 + lane16)
                    rev = 15 - lane16
                    ckr = ck[rev]
                    cvr = cv[rev]
                    bd = bdv[...]
                    bi = biv[...]
                    takeb = bd <= ckr
                    kk = jnp.where(takeb, bd, ckr)
                    vv = jnp.where(takeb, bi, cvr)
                    for s in (8, 4, 2, 1):
                        prt = jnp.bitwise_xor(lane16, s)
                        kb = kk[prt]
                        vb = vv[prt]
                        lt = lex_lt_i(kk, kb, vv, vb)
                        keep = (b2i((lane16 & s) == 0) ^ lt) == 0
                        kk = jnp.where(keep, kk, kb)
                        vv = jnp.where(keep, vv, vb)
                    bdv[...] = kk
                    biv[...] = vv

                return carry

            bdv[...] = jnp.full((16,), jnp.inf, jnp.float32)
            biv[...] = jnp.zeros((16,), jnp.int32)
            lax.fori_loop(0, nchunk, chunk_body, 0)
            ov[pl.ds(rl * K, K)] = biv[...] + goff
            return carry

        lax.fori_loop(0, rpw, row_body, 0)
        pltpu.sync_copy(ov, out_hbm.at[pl.ds(wid * rpw * K, rpw * K)])

    return sc_kernel(xb, yb, zb, sq, )


def kernel(points):
    B, N, C = points.shape
    parts = []
    if SCR > 0:
        pts_sc = points[B - 1]
        u = jax.lax.bitcast_convert_type(pts_sc, jnp.uint32)
        rnd = ((u >> 16) & jnp.uint32(1)) + jnp.uint32(0x7FFF)
        pb = jax.lax.bitcast_convert_type(
            (u + rnd) & jnp.uint32(0xFFFF0000), jnp.float32)
        sq = jnp.sum(pts_sc * pts_sc, axis=-1)
        sc_dst = _sc_knn(pb[:, 0], pb[:, 1], pb[:, 2], sq, B - 1, SCR, N)
    nblk = (B * N - SCR) // BQ
    tc_dst = _tc_knn(points, nblk).reshape(-1)[:nblk * BQ * K]
    parts.append(tc_dst)
    if SCR > 0:
        parts.append(sc_dst)
    dst = jnp.concatenate(parts) if len(parts) > 1 else parts[0]
    src = jnp.broadcast_to(
        jnp.arange(B * N, dtype=jnp.int32).reshape(B * N, 1), (B * N, K))
    return jnp.stack([src.reshape(-1), dst.reshape(-1)], axis=0)


# speedup vs baseline: 1.0572x; 1.0572x over previous
import functools

import jax
import jax.numpy as jnp
from jax import lax
from jax.experimental import pallas as pl
from jax.experimental.pallas import tpu as pltpu
from jax.experimental.pallas import tpu_sc as plsc

K = 16
BQ = 256
CPAD = 8
SCR = 1792
NW = 32


def _knn_block(pts_nc_ref, pts_cn_ref, idx_ref, *, n):
    b = pl.program_id(0) // (n // BQ)
    q = pts_nc_ref[0]
    p = pts_cn_ref[0]
    sq_q = jnp.sum(q * q, axis=1, keepdims=True)
    sq_p = jnp.sum(p * p, axis=0, keepdims=True)
    inner = jax.lax.dot_general(
        q, p, (((1,), (0,)), ((), ())),
        preferred_element_type=jnp.float32)
    d = sq_q + sq_p - 2.0 * inner
    iota = jax.lax.broadcasted_iota(jnp.int32, (BQ, n), 1).astype(jnp.float32)
    nf = jnp.float32(n)
    cols = []
    for _ in range(K):
        m = jnp.min(d, axis=1, keepdims=True)
        cand = jnp.where(d <= m, iota, nf)
        a = jnp.min(cand, axis=1, keepdims=True)
        cols.append(a)
        d = jnp.where(iota == a, jnp.float32(jnp.inf), d)
    idx = jnp.concatenate(cols, axis=1).astype(jnp.int32)
    idx_ref[0] = idx + b * n


def _tc_knn(points, nblk):
    B, N, C = points.shape
    bpb = N // BQ
    pts_nc = jnp.pad(points, ((0, 0), (0, 0), (0, CPAD - C)))
    pts_cn = jnp.transpose(pts_nc, (0, 2, 1))
    return pl.pallas_call(
        functools.partial(_knn_block, n=N),
        grid=(nblk,),
        in_specs=[
            pl.BlockSpec((1, BQ, CPAD), lambda t: (t // bpb, t % bpb, 0)),
            pl.BlockSpec((1, CPAD, N), lambda t: (t // bpb, 0, 0)),
        ],
        out_specs=pl.BlockSpec((1, BQ, K), lambda t: (t // bpb, t % bpb, 0)),
        out_shape=jax.ShapeDtypeStruct((B, N, K), jnp.int32),
    )(pts_nc, pts_cn)


def _sc_knn(xb, yb, zb, sq, b0, scr, n):
    rpw = scr // NW
    nchunk = n // 16
    mesh = plsc.VectorSubcoreMesh(core_axis_name="c", subcore_axis_name="s")

    @functools.partial(
        pl.kernel, mesh=mesh,
        out_type=jax.ShapeDtypeStruct((scr * K,), jnp.int32),
        scratch_types=[
            pltpu.VMEM((n,), jnp.float32),
            pltpu.VMEM((n,), jnp.float32),
            pltpu.VMEM((n,), jnp.float32),
            pltpu.VMEM((n,), jnp.float32),
            pltpu.VMEM((rpw * K,), jnp.int32),
            pltpu.VMEM((K,), jnp.float32),
            pltpu.VMEM((K,), jnp.int32),
        ],
    )
    def sc_kernel(xb_hbm, yb_hbm, zb_hbm, sq_hbm, out_hbm,
                  xv, yv, zv, sv, ov, bdv, biv):
        wid = lax.axis_index("s") * 2 + lax.axis_index("c")
        pltpu.sync_copy(xb_hbm, xv)
        pltpu.sync_copy(yb_hbm, yv)
        pltpu.sync_copy(zb_hbm, zv)
        pltpu.sync_copy(sq_hbm, sv)
        r0 = (n - scr) + wid * rpw
        goff = b0 * n

        def row_body(rl, carry):
            r = r0 + rl
            g16 = pl.multiple_of((r // 16) * 16, 16)
            lane16 = lax.iota(jnp.int32, 16)
            lane = jnp.broadcast_to(r - g16, (16,))
            qx = xv[pl.ds(g16, 16)][lane]
            qy = yv[pl.ds(g16, 16)][lane]
            qz = zv[pl.ds(g16, 16)][lane]
            sqq = sv[pl.ds(g16, 16)][lane]

            ones = jnp.broadcast_to(1, (16,))
            zeros = jnp.broadcast_to(0, (16,))

            def b2i(m):
                return jnp.where(m, ones, zeros)

            def lex_lt_i(kk, kb, vv, vb):
                return b2i(kk < kb) | (b2i(kk == kb) & b2i(vv < vb))

            def sort16(kk, vv):
                for kbit in (2, 4, 8, 16):
                    s = kbit // 2
                    while s >= 1:
                        prt = jnp.bitwise_xor(lane16, s)
                        kb = kk[prt]
                        vb = vv[prt]
                        lt = lex_lt_i(kk, kb, vv, vb)
                        mm = ones ^ b2i((lane16 & s) == 0) ^ b2i((lane16 & kbit) == 0)
                        keep = (mm ^ lt) == 0
                        kk = jnp.where(keep, kk, kb)
                        vv = jnp.where(keep, vv, vb)
                        s //= 2
                return kk, vv

            def chunk_body(j, carry):
                o = pl.multiple_of(j * 16, 16)
                px = xv[pl.ds(o, 16)]
                py = yv[pl.ds(o, 16)]
                pz = zv[pl.ds(o, 16)]
                sp = sv[pl.ds(o, 16)]
                inner = (qx * px + qy * py) + qz * pz
                d = (sqq + sp) - 2.0 * inner
                mn = d
                for s in (8, 4, 2, 1):
                    mn = jnp.minimum(mn, mn[jnp.bitwise_xor(lane16, s)])

                @pl.when(mn[0] < bdv[...][15])
                def _merge():
                    ck, cv = sort16(d, j * 16 + lane16)
                    rev = 15 - lane16
                    ckr = ck[rev]
                    cvr = cv[rev]
                    bd = bdv[...]
                    bi = biv[...]
                    takeb = bd <= ckr
                    kk = jnp.where(takeb, bd, ckr)
                    vv = jnp.where(takeb, bi, cvr)
                    for s in (8, 4, 2, 1):
                        prt = jnp.bitwise_xor(lane16, s)
                        kb = kk[prt]
                        vb = vv[prt]
                        lt = lex_lt_i(kk, kb, vv, vb)
                        keep = (b2i((lane16 & s) == 0) ^ lt) == 0
                        kk = jnp.where(keep, kk, kb)
                        vv = jnp.where(keep, vv, vb)
                    bdv[...] = kk
                    biv[...] = vv

                return carry

            bdv[...] = jnp.full((16,), jnp.inf, jnp.float32)
            biv[...] = jnp.zeros((16,), jnp.int32)
            lax.fori_loop(0, nchunk, chunk_body, 0)
            ov[pl.ds(rl * K, K)] = biv[...] + goff
            return carry

        lax.fori_loop(0, rpw, row_body, 0)
        pltpu.sync_copy(ov, out_hbm.at[pl.ds(wid * rpw * K, rpw * K)])

    return sc_kernel(xb, yb, zb, sq, )


def kernel(points):
    B, N, C = points.shape
    parts = []
    if SCR > 0:
        pts_sc = points[B - 1]
        u = jax.lax.bitcast_convert_type(pts_sc, jnp.uint32)
        rnd = ((u >> 16) & jnp.uint32(1)) + jnp.uint32(0x7FFF)
        pb = jax.lax.bitcast_convert_type(
            (u + rnd) & jnp.uint32(0xFFFF0000), jnp.float32)
        sq = jnp.sum(pts_sc * pts_sc, axis=-1)
        sc_dst = _sc_knn(pb[:, 0], pb[:, 1], pb[:, 2], sq, B - 1, SCR, N)
    nblk = (B * N - SCR) // BQ
    tc_dst = _tc_knn(points, nblk).reshape(-1)[:nblk * BQ * K]
    parts.append(tc_dst)
    if SCR > 0:
        parts.append(sc_dst)
    dst = jnp.concatenate(parts) if len(parts) > 1 else parts[0]
    src = jnp.broadcast_to(
        jnp.arange(B * N, dtype=jnp.int32).reshape(B * N, 1), (B * N, K))
    return jnp.stack([src.reshape(-1), dst.reshape(-1)], axis=0)


# speedup vs baseline: 1.0589x; 1.0017x over previous
import functools

import jax
import jax.numpy as jnp
from jax import lax
from jax.experimental import pallas as pl
from jax.experimental.pallas import tpu as pltpu
from jax.experimental.pallas import tpu_sc as plsc

K = 16
BQ = 512
CPAD = 8
SCR = 1536
NW = 32


def _knn_block(pts_nc_ref, pts_cn_ref, idx_ref, *, n):
    b = pl.program_id(0) // (n // BQ)
    q = pts_nc_ref[0]
    p = pts_cn_ref[0]
    sq_q = jnp.sum(q * q, axis=1, keepdims=True)
    sq_p = jnp.sum(p * p, axis=0, keepdims=True)
    inner = jax.lax.dot_general(
        q, p, (((1,), (0,)), ((), ())),
        preferred_element_type=jnp.float32)
    d = sq_q + sq_p - 2.0 * inner
    iota = jax.lax.broadcasted_iota(jnp.int32, (BQ, n), 1).astype(jnp.float32)
    nf = jnp.float32(n)
    cols = []
    for _ in range(K):
        m = jnp.min(d, axis=1, keepdims=True)
        cand = jnp.where(d <= m, iota, nf)
        a = jnp.min(cand, axis=1, keepdims=True)
        cols.append(a)
        d = jnp.where(iota == a, jnp.float32(jnp.inf), d)
    idx = jnp.concatenate(cols, axis=1).astype(jnp.int32)
    idx_ref[0] = idx + b * n


def _tc_knn(points, nblk):
    B, N, C = points.shape
    bpb = N // BQ
    pts_nc = jnp.pad(points, ((0, 0), (0, 0), (0, CPAD - C)))
    pts_cn = jnp.transpose(pts_nc, (0, 2, 1))
    return pl.pallas_call(
        functools.partial(_knn_block, n=N),
        grid=(nblk,),
        in_specs=[
            pl.BlockSpec((1, BQ, CPAD), lambda t: (t // bpb, t % bpb, 0)),
            pl.BlockSpec((1, CPAD, N), lambda t: (t // bpb, 0, 0)),
        ],
        out_specs=pl.BlockSpec((1, BQ, K), lambda t: (t // bpb, t % bpb, 0)),
        out_shape=jax.ShapeDtypeStruct((B, N, K), jnp.int32),
    )(pts_nc, pts_cn)


def _sc_knn(xb, yb, zb, sq, b0, scr, n):
    rpw = scr // NW
    nchunk = n // 16
    mesh = plsc.VectorSubcoreMesh(core_axis_name="c", subcore_axis_name="s")

    @functools.partial(
        pl.kernel, mesh=mesh,
        out_type=jax.ShapeDtypeStruct((scr * K,), jnp.int32),
        scratch_types=[
            pltpu.VMEM((n,), jnp.float32),
            pltpu.VMEM((n,), jnp.float32),
            pltpu.VMEM((n,), jnp.float32),
            pltpu.VMEM((n,), jnp.float32),
            pltpu.VMEM((rpw * K,), jnp.int32),
            pltpu.VMEM((K,), jnp.float32),
            pltpu.VMEM((K,), jnp.int32),
        ],
    )
    def sc_kernel(xb_hbm, yb_hbm, zb_hbm, sq_hbm, out_hbm,
                  xv, yv, zv, sv, ov, bdv, biv):
        wid = lax.axis_index("s") * 2 + lax.axis_index("c")
        pltpu.sync_copy(xb_hbm, xv)
        pltpu.sync_copy(yb_hbm, yv)
        pltpu.sync_copy(zb_hbm, zv)
        pltpu.sync_copy(sq_hbm, sv)
        r0 = (n - scr) + wid * rpw
        goff = b0 * n

        def row_body(rl, carry):
            r = r0 + rl
            g16 = pl.multiple_of((r // 16) * 16, 16)
            lane16 = lax.iota(jnp.int32, 16)
            lane = jnp.broadcast_to(r - g16, (16,))
            qx = xv[pl.ds(g16, 16)][lane]
            qy = yv[pl.ds(g16, 16)][lane]
            qz = zv[pl.ds(g16, 16)][lane]
            sqq = sv[pl.ds(g16, 16)][lane]

            ones = jnp.broadcast_to(1, (16,))
            zeros = jnp.broadcast_to(0, (16,))

            def b2i(m):
                return jnp.where(m, ones, zeros)

            def lex_lt_i(kk, kb, vv, vb):
                return b2i(kk < kb) | (b2i(kk == kb) & b2i(vv < vb))

            def sort16(kk, vv):
                for kbit in (2, 4, 8, 16):
                    s = kbit // 2
                    while s >= 1:
                        prt = jnp.bitwise_xor(lane16, s)
                        kb = kk[prt]
                        vb = vv[prt]
                        lt = lex_lt_i(kk, kb, vv, vb)
                        mm = ones ^ b2i((lane16 & s) == 0) ^ b2i((lane16 & kbit) == 0)
                        keep = (mm ^ lt) == 0
                        kk = jnp.where(keep, kk, kb)
                        vv = jnp.where(keep, vv, vb)
                        s //= 2
                return kk, vv

            def chunk_body(j, carry):
                o = pl.multiple_of(j * 16, 16)
                px = xv[pl.ds(o, 16)]
                py = yv[pl.ds(o, 16)]
                pz = zv[pl.ds(o, 16)]
                sp = sv[pl.ds(o, 16)]
                inner = (qx * px + qy * py) + qz * pz
                d = (sqq + sp) - 2.0 * inner
                mn = d
                for s in (8, 4, 2, 1):
                    mn = jnp.minimum(mn, mn[jnp.bitwise_xor(lane16, s)])

                @pl.when(mn[0] < bdv[...][15])
                def _merge():
                    ck, cv = sort16(d, j * 16 + lane16)
                    rev = 15 - lane16
                    ckr = ck[rev]
                    cvr = cv[rev]
                    bd = bdv[...]
                    bi = biv[...]
                    takeb = bd <= ckr
                    kk = jnp.where(takeb, bd, ckr)
                    vv = jnp.where(takeb, bi, cvr)
                    for s in (8, 4, 2, 1):
                        prt = jnp.bitwise_xor(lane16, s)
                        kb = kk[prt]
                        vb = vv[prt]
                        lt = lex_lt_i(kk, kb, vv, vb)
                        keep = (b2i((lane16 & s) == 0) ^ lt) == 0
                        kk = jnp.where(keep, kk, kb)
                        vv = jnp.where(keep, vv, vb)
                    bdv[...] = kk
                    biv[...] = vv

                return carry

            bdv[...] = jnp.full((16,), jnp.inf, jnp.float32)
            biv[...] = jnp.zeros((16,), jnp.int32)
            lax.fori_loop(0, nchunk, chunk_body, 0)
            ov[pl.ds(rl * K, K)] = biv[...] + goff
            return carry

        lax.fori_loop(0, rpw, row_body, 0)
        pltpu.sync_copy(ov, out_hbm.at[pl.ds(wid * rpw * K, rpw * K)])

    return sc_kernel(xb, yb, zb, sq, )


def kernel(points):
    B, N, C = points.shape
    parts = []
    if SCR > 0:
        pts_sc = points[B - 1]
        u = jax.lax.bitcast_convert_type(pts_sc, jnp.uint32)
        rnd = ((u >> 16) & jnp.uint32(1)) + jnp.uint32(0x7FFF)
        pb = jax.lax.bitcast_convert_type(
            (u + rnd) & jnp.uint32(0xFFFF0000), jnp.float32)
        sq = jnp.sum(pts_sc * pts_sc, axis=-1)
        sc_dst = _sc_knn(pb[:, 0], pb[:, 1], pb[:, 2], sq, B - 1, SCR, N)
    nblk = (B * N - SCR) // BQ
    tc_dst = _tc_knn(points, nblk).reshape(-1)[:nblk * BQ * K]
    parts.append(tc_dst)
    if SCR > 0:
        parts.append(sc_dst)
    dst = jnp.concatenate(parts) if len(parts) > 1 else parts[0]
    src = jnp.broadcast_to(
        jnp.arange(B * N, dtype=jnp.int32).reshape(B * N, 1), (B * N, K))
    return jnp.stack([src.reshape(-1), dst.reshape(-1)], axis=0)
